# baseline (speedup 1.0000x reference)
import functools

import jax
import jax.numpy as jnp
from jax import lax
from jax.experimental import pallas as pl
from jax.experimental.pallas import tpu as pltpu
from jax.experimental.pallas import tpu_sc as plsc

NBUF = 2
NC, NS = 2, 16
NW = NC * NS

_mesh = plsc.VectorSubcoreMesh(core_axis_name="c", subcore_axis_name="s")


def _make_gather(batch: int, hist: int, d_model: int):
    assert batch % NW == 0
    b_per_w = batch // NW
    assert b_per_w % NBUF == 0
    w0 = (hist // 2 + 7) // 8 * 8
    w1 = hist - w0
    assert 0 < w1 <= 128 and w0 <= 128 and w0 % 8 == 0

    @functools.partial(
        pl.kernel,
        mesh=_mesh,
        compiler_params=pltpu.CompilerParams(use_tc_tiling_on_sc=False),
        out_type=jax.ShapeDtypeStruct((batch, hist, d_model), jnp.float32),
        scratch_types=[
            pltpu.VMEM((b_per_w, hist), jnp.int32),
            pltpu.VMEM((NBUF, hist, d_model), jnp.float32),
            pltpu.SemaphoreType.DMA,
            pltpu.SemaphoreType.DMA,
        ],
    )
    def _gather(x_hbm, table_hbm, out_hbm, idx_v, rows_v, sem0, sem1):
        wid = lax.axis_index("s") * NC + lax.axis_index("c")
        b0 = wid * b_per_w
        sems = [sem0, sem1]

        pltpu.sync_copy(x_hbm.at[pl.ds(b0, b_per_w)], idx_v)

        def fire(r, buf):
            pltpu.async_copy(
                table_hbm.at[idx_v.at[r, pl.ds(0, w0)]],
                rows_v.at[buf].at[pl.ds(0, w0)],
                sems[buf],
            )
            pltpu.async_copy(
                table_hbm.at[idx_v.at[r, pl.ds(w0, w1)]],
                rows_v.at[buf].at[pl.ds(w0, w1)],
                sems[buf],
            )

        def drain(buf):
            pltpu.make_async_copy(
                out_hbm.at[0], rows_v.at[buf], sems[buf]
            ).wait()

        fire(0, 0)

        def pair(p, carry):
            r0 = p * NBUF
            for b in range(NBUF):
                r = r0 + b

                @pl.when(r + 1 < b_per_w)
                def _():
                    fire(r + 1, (b + 1) % NBUF)

                drain(b)
                pltpu.sync_copy(rows_v.at[b], out_hbm.at[b0 + r])
            return carry

        lax.fori_loop(0, b_per_w // NBUF, pair, 0)

    return _gather


def kernel(x, table):
    b, h = x.shape
    return _make_gather(b, h, table.shape[1])(x.astype(jnp.int32), table)


# speedup vs baseline: 1.0031x; 1.0031x over previous
import functools

import jax
import jax.numpy as jnp
from jax import lax
from jax.experimental import pallas as pl
from jax.experimental.pallas import tpu as pltpu
from jax.experimental.pallas import tpu_sc as plsc

GATHER_W = 128
K = 5
NBUF = 2
NC, NS = 2, 16
NW = NC * NS

_mesh = plsc.VectorSubcoreMesh(core_axis_name="c", subcore_axis_name="s")


def _make_gather(n: int, d_model: int):
    rows_per_w = n // NW
    chunk = K * GATHER_W
    assert rows_per_w % chunk == 0
    n_chunks = rows_per_w // chunk
    assert n_chunks % NBUF == 0

    @functools.partial(
        pl.kernel,
        mesh=_mesh,
        compiler_params=pltpu.CompilerParams(use_tc_tiling_on_sc=False),
        out_type=jax.ShapeDtypeStruct((n, d_model), jnp.float32),
        scratch_types=[
            pltpu.VMEM((rows_per_w,), jnp.int32),
            pltpu.VMEM((NBUF, chunk, d_model), jnp.float32),
            pltpu.SemaphoreType.DMA,
            pltpu.SemaphoreType.DMA,
        ],
    )
    def _gather(idx_hbm, table_hbm, out_hbm, idx_v, rows_v, sem0, sem1):
        wid = lax.axis_index("s") * NC + lax.axis_index("c")
        base = wid * rows_per_w
        sems = [sem0, sem1]

        pltpu.sync_copy(idx_hbm.at[pl.ds(base, rows_per_w)], idx_v)

        def fire(c, buf):
            for j in range(K):
                pltpu.async_copy(
                    table_hbm.at[idx_v.at[pl.ds(c * chunk + j * GATHER_W, GATHER_W)]],
                    rows_v.at[buf].at[pl.ds(j * GATHER_W, GATHER_W)],
                    sems[buf],
                )

        def drain(buf):
            pltpu.make_async_copy(
                out_hbm.at[pl.ds(0, chunk)], rows_v.at[buf], sems[buf]
            ).wait()

        fire(0, 0)

        def pair(p, carry):
            c0 = p * NBUF
            for b in range(NBUF):
                c = c0 + b

                @pl.when(c + 1 < n_chunks)
                def _():
                    fire(c + 1, (b + 1) % NBUF)

                drain(b)
                pltpu.sync_copy(
                    rows_v.at[b], out_hbm.at[pl.ds(base + c * chunk, chunk)]
                )
            return carry

        lax.fori_loop(0, n_chunks // NBUF, pair, 0)

    return _gather


def kernel(x, table):
    b, h = x.shape
    flat = x.reshape(-1).astype(jnp.int32)
    out = _make_gather(flat.shape[0], table.shape[1])(flat, table)
    return out.reshape(b, h, table.shape[1])
